# baseline (speedup 1.0000x reference)
import functools

import jax
import jax.numpy as jnp
from jax.experimental import pallas as pl
from jax.experimental.pallas import tpu as pltpu
from jax.experimental.pallas import tpu_sc as plsc

_N = 2048
_NBIN = 64
_MPB = 256
_CUT = 0.2
_TILE = 256
_NT = _N // _TILE
_NCLS = 12
_D1 = 512
_D2 = 520

_SELU_SCALE = 1.0507009873554805
_SELU_ALPHA = 1.6732632423543772


def _selu(x):
    return _SELU_SCALE * jnp.where(x > 0, x, _SELU_ALPHA * (jnp.exp(x) - 1.0))


def _mm(a, b):
    return jnp.dot(a, b, preferred_element_type=jnp.float32)


def _dotT(a, b):
    return jax.lax.dot_general(
        a, b, (((1,), (1,)), ((), ())), preferred_element_type=jnp.float32)


def _encode(x):
    ids = x[:, 0:1].astype(jnp.int32)
    ioh = jax.lax.broadcasted_iota(jnp.int32, (_N, _NCLS), 1)
    oh = (ids == ioh).astype(jnp.float32)
    return jnp.concatenate([oh, x[:, 1:]], axis=1)


def _prep_body(x_ref, wd_ref, bd_ref, W1, b1, W2, b2, W3, b3,
               c1Wt, c1bt, c1Wh, c1th,
               pts_ref, lsh_ref, fh_ref, gate_ref, base_ref):
    enc = _encode(x_ref[0])
    xd = _selu(_mm(enc, wd_ref[...]) + bd_ref[...])
    lsh_ref[0] = xd[:, :_NBIN]
    pts_ref[0] = xd[:, _NBIN:]
    x = _selu(_mm(enc, W1[...]) + b1[...])
    x = _selu(_mm(x, W2[...]) + b2[...])
    x = _selu(_mm(x, W3[...]) + b3[...])
    gate = jax.nn.sigmoid(_mm(x, c1Wt[...]) + c1bt[...])
    fh_ref[0] = _mm(x, c1th[...])
    gate_ref[0] = gate
    base_ref[0] = (1.0 - gate) * _mm(x, c1Wh[...])


_CHUNK = 128
_NGRP = _CHUNK // 16


def _route_body(lsh_hbm, m_hbm, lsh_v, cnt_v, offs_v, m_v, csh, acf):
    sample = jax.lax.axis_index("c")
    s = jax.lax.axis_index("s")
    base = sample * _N + s * _CHUNK
    pltpu.sync_copy(lsh_hbm.at[pl.ds(base * _NBIN, _CHUNK * _NBIN)], lsh_v)
    zeros16f = jnp.zeros((16,), jnp.float32)
    for q in range(_NBIN // 16):
        cnt_v[q * 16:(q + 1) * 16] = zeros16f
    iota16 = jax.lax.broadcasted_iota(jnp.int32, (16,), 0)
    lane0 = iota16 == 0
    bins_all = []
    ranks_all = []
    for g in range(_NGRP):
        rowoff = (iota16 + g * 16) * _NBIN
        v0 = plsc.load_gather(lsh_v, [rowoff])

        def cbody(c, carry, rowoff=rowoff):
            cur_max, cur_bin = carry
            cvec = jnp.zeros((16,), jnp.int32) + c
            v = plsc.load_gather(lsh_v, [rowoff + cvec])
            upd = v > cur_max
            return (jnp.where(upd, v, cur_max), jnp.where(upd, cvec, cur_bin))

        _, bins = jax.lax.fori_loop(
            1, _NBIN, cbody, (v0, jnp.zeros((16,), jnp.int32)))

        def bbody(b, prev, bins=bins):
            bvec = jnp.zeros((16,), jnp.int32) + b
            mask = bins == bvec
            mi = mask.astype(jnp.int32)
            pc = plsc.cumsum(mi)
            cntb = plsc.load_gather(cnt_v, [bvec])
            prev = jnp.where(
                mask, cntb + (pc - mi).astype(jnp.float32), prev)
            tot = plsc.all_reduce_population_count(mask)
            plsc.store_scatter(
                cnt_v, [bvec], cntb + tot.astype(jnp.float32), mask=lane0)
            return prev

        rank = jax.lax.fori_loop(0, _NBIN, bbody, zeros16f)
        bins_all.append(bins)
        ranks_all.append(rank)
    pltpu.sync_copy(cnt_v, csh.at[pl.ds(s * _NBIN, _NBIN)])
    plsc.subcore_barrier()
    pltpu.sync_copy(csh, acf)
    for q in range(_NBIN // 16):
        acc = zeros16f
        for j in range(16):
            f = jnp.where(j < s, 1.0, 0.0)
            acc = acc + f * acf[j * _NBIN + q * 16:j * _NBIN + (q + 1) * 16]
        offs_v[q * 16:(q + 1) * 16] = acc
    for g in range(_NGRP):
        off = plsc.load_gather(offs_v, [bins_all[g]])
        grank = ranks_all[g] + off
        mval = jnp.where(
            grank < float(_MPB), bins_all[g].astype(jnp.float32), -1.0)
        m_v[g * 16:(g + 1) * 16] = mval
    pltpu.sync_copy(m_v, m_hbm.at[pl.ds(base, _CHUNK)])


def _make_route(nb):
    return pl.kernel(
        _route_body,
        out_type=jax.ShapeDtypeStruct((nb * _N,), jnp.float32),
        mesh=plsc.VectorSubcoreMesh(
            core_axis_name="c", subcore_axis_name="s", num_cores=nb),
        scratch_types=[
            pltpu.VMEM((_CHUNK * _NBIN,), jnp.float32),
            pltpu.VMEM((_NBIN,), jnp.float32),
            pltpu.VMEM((_NBIN,), jnp.float32),
            pltpu.VMEM((_CHUNK,), jnp.float32),
            pltpu.VMEM_SHARED((16 * _NBIN,), jnp.float32),
            pltpu.VMEM((16 * _NBIN,), jnp.float32),
        ],
        compiler_params=pltpu.CompilerParams(needs_layout_passes=False),
    )


def _adj_body(ptsr_ref, mr_ref, pts_ref, mf_ref, adj_ref, deg_ref):
    rows = ptsr_ref[0]
    ptsf = pts_ref[0]
    mr = mr_ref[0]
    mf = mf_ref[0]
    ior = jax.lax.broadcasted_iota(
        jnp.int32, (_TILE, _NBIN), 1).astype(jnp.float32)
    iof = jax.lax.broadcasted_iota(
        jnp.int32, (_N, _NBIN), 1).astype(jnp.float32)
    bvr = (mr == ior).astype(jnp.float32)
    bvf = (mf == iof).astype(jnp.float32)
    vm = _dotT(bvr, bvf) > 0.5
    sim = _dotT(rows, ptsf)
    sim = jnp.where(vm, sim, -1e9)
    m = jnp.max(sim, axis=1, keepdims=True)
    e = jnp.exp(sim - m)
    d = e / jnp.sum(e, axis=1, keepdims=True)
    d = jnp.where(vm, d, 0.0)
    a = jnp.where(d > _CUT, jnp.exp(-d), 0.0)
    adj_ref[0] = a
    deg_ref[0] = jnp.sum(a, axis=1, keepdims=True)


def _gh_body(adj_ref, fh_ref, gate_ref, base_ref, degf_ref, degr_ref,
             out_ref):
    nf = jax.lax.rsqrt(degf_ref[0] + 1e-6)
    nr = jax.lax.rsqrt(degr_ref[0] + 1e-6)
    agg = _mm(adj_ref[0], fh_ref[0] * nf)
    out_ref[0] = gate_ref[0] * nr * agg + base_ref[0]


def _head_body(xg_ref,
               id1W, id1b, id2W, id2b, id3W, id3b,
               idW, idb, chW, chb,
               c2Wt, c2bt, c2Wh, c2th,
               lc_ref, fh_ref, gate_ref, base_ref):
    xg = xg_ref[0]
    a = _selu(_mm(xg, id1W[...]) + id1b[...])
    a = _selu(_mm(a, id2W[...]) + id2b[...])
    a = _selu(_mm(a, id3W[...]) + id3b[...])
    logits = _mm(a, idW[...]) + idb[...]
    charge = _mm(a, chW[...]) + chb[...]
    lm = jnp.max(logits, axis=1, keepdims=True)
    le = jnp.exp(logits - lm)
    sm = le / jnp.sum(le, axis=1, keepdims=True)
    x2 = jnp.concatenate([xg, sm], axis=1)
    gate = jax.nn.sigmoid(_mm(x2, c2Wt[...]) + c2bt[...])
    lc_ref[0] = jnp.concatenate([logits, charge], axis=1)
    fh_ref[0] = _mm(x2, c2th[...])
    gate_ref[0] = gate
    base_ref[0] = (1.0 - gate) * _mm(x2, c2Wh[...])


def _tail_body(x_ref, x2g_ref, lc_ref,
               m1W, m1b, m2W, m2b, m3W, m3b,
               mo1W, mo1b, mo2W, mo2b, mo3W, mo3b, moW, mob,
               out_ref):
    enc = _encode(x_ref[0])
    xm = _selu(_mm(enc, m1W[...]) + m1b[...])
    xm = _selu(_mm(xm, m2W[...]) + m2b[...])
    xm = _selu(_mm(xm, m3W[...]) + m3b[...])
    d = jnp.concatenate([x2g_ref[0], xm], axis=1)
    d = _selu(_mm(d, mo1W[...]) + mo1b[...])
    d = _selu(_mm(d, mo2W[...]) + mo2b[...])
    d = _selu(_mm(d, mo3W[...]) + mo3b[...])
    mom = _mm(d, moW[...]) + mob[...]
    out_ref[0] = jnp.concatenate([lc_ref[0], mom], axis=1)


def _wspec(arr):
    return pl.BlockSpec(arr.shape, lambda *_: (0,) * arr.ndim)


def _full(d):
    return pl.BlockSpec((1, _N, d), lambda b: (b, 0, 0))


def kernel(X, params):
    p = params
    B = X.shape[0]
    f32 = jnp.float32

    def rb(v):
        return v.reshape(1, -1)

    def shp(d):
        return jax.ShapeDtypeStruct((B, _N, d), f32)

    w1 = [p['W1'], rb(p['b1']), p['W2'], rb(p['b2']), p['W3'], rb(p['b3']),
          p['c1_Wt'], rb(p['c1_bt']), p['c1_Wh'], p['c1_theta']]
    pts, lsh, fh1, gate1, base1 = pl.pallas_call(
        _prep_body,
        grid=(B,),
        in_specs=[_full(15), _wspec(p['Wd']),
                  pl.BlockSpec((1, 256), lambda b: (0, 0))]
        + [_wspec(w) for w in w1],
        out_specs=[_full(192), _full(_NBIN), _full(_D1), _full(_D1),
                   _full(_D1)],
        out_shape=[shp(192), shp(_NBIN), shp(_D1), shp(_D1), shp(_D1)],
    )(X, p['Wd'], rb(p['bd']), *w1)

    m = _make_route(B)(lsh.reshape(B * _N * _NBIN))
    m3 = m.reshape(B, _N, 1)

    adj, deg = pl.pallas_call(
        _adj_body,
        grid=(B, _NT),
        in_specs=[
            pl.BlockSpec((1, _TILE, 192), lambda b, t: (b, t, 0)),
            pl.BlockSpec((1, _TILE, 1), lambda b, t: (b, t, 0)),
            pl.BlockSpec((1, _N, 192), lambda b, t: (b, 0, 0)),
            pl.BlockSpec((1, _N, 1), lambda b, t: (b, 0, 0)),
        ],
        out_specs=[
            pl.BlockSpec((1, _TILE, _N), lambda b, t: (b, t, 0)),
            pl.BlockSpec((1, _TILE, 1), lambda b, t: (b, t, 0)),
        ],
        out_shape=[shp(_N), shp(1)],
    )(pts, m3, pts, m3)

    def gh(adj, fh, gate, base, deg, d):
        return pl.pallas_call(
            _gh_body,
            grid=(B, _NT),
            in_specs=[
                pl.BlockSpec((1, _TILE, _N), lambda b, t: (b, t, 0)),
                pl.BlockSpec((1, _N, d), lambda b, t: (b, 0, 0)),
                pl.BlockSpec((1, _TILE, d), lambda b, t: (b, t, 0)),
                pl.BlockSpec((1, _TILE, d), lambda b, t: (b, t, 0)),
                pl.BlockSpec((1, _N, 1), lambda b, t: (b, 0, 0)),
                pl.BlockSpec((1, _TILE, 1), lambda b, t: (b, t, 0)),
            ],
            out_specs=pl.BlockSpec((1, _TILE, d), lambda b, t: (b, t, 0)),
            out_shape=shp(d),
        )(adj, fh, gate, base, deg, deg)

    xg = gh(adj, fh1, gate1, base1, deg, _D1)

    w2 = [p['id1_W'], rb(p['id1_b']), p['id2_W'], rb(p['id2_b']),
          p['id3_W'], rb(p['id3_b']), p['idW'], rb(p['idb']),
          p['chW'], rb(p['chb']),
          p['c2_Wt'], rb(p['c2_bt']), p['c2_Wh'], p['c2_theta']]
    lc, fh2, gate2, base2 = pl.pallas_call(
        _head_body,
        grid=(B,),
        in_specs=[_full(_D1)] + [_wspec(w) for w in w2],
        out_specs=[_full(9), _full(_D2), _full(_D2), _full(_D2)],
        out_shape=[shp(9), shp(_D2), shp(_D2), shp(_D2)],
    )(xg, *w2)

    x2g = gh(adj, fh2, gate2, base2, deg, _D2)

    w3 = [p['m1_W'], rb(p['m1_b']), p['m2_W'], rb(p['m2_b']),
          p['m3_W'], rb(p['m3_b']),
          p['mo1_W'], rb(p['mo1_b']), p['mo2_W'], rb(p['mo2_b']),
          p['mo3_W'], rb(p['mo3_b']), p['moW'], rb(p['mob'])]
    out = pl.pallas_call(
        _tail_body,
        grid=(B,),
        in_specs=[_full(15), _full(_D2), _full(9)] + [_wspec(w) for w in w3],
        out_specs=_full(12),
        out_shape=shp(12),
    )(X, x2g, lc, *w3)
    return out


# speedup vs baseline: 1.0553x; 1.0553x over previous
import functools

import jax
import jax.numpy as jnp
from jax.experimental import pallas as pl
from jax.experimental.pallas import tpu as pltpu
from jax.experimental.pallas import tpu_sc as plsc

_N = 2048
_NBIN = 64
_MPB = 256
_CUT = 0.2
_TILE = 256
_NT = _N // _TILE
_NCLS = 12
_D1 = 512
_D2 = 520

_SELU_SCALE = 1.0507009873554805
_SELU_ALPHA = 1.6732632423543772


def _selu(x):
    return _SELU_SCALE * jnp.where(x > 0, x, _SELU_ALPHA * (jnp.exp(x) - 1.0))


def _mm(a, b):
    return jnp.dot(a, b, preferred_element_type=jnp.float32)


def _dotT(a, b):
    return jax.lax.dot_general(
        a, b, (((1,), (1,)), ((), ())), preferred_element_type=jnp.float32)


def _encode(x):
    ids = x[:, 0:1].astype(jnp.int32)
    ioh = jax.lax.broadcasted_iota(jnp.int32, (_N, _NCLS), 1)
    oh = (ids == ioh).astype(jnp.float32)
    return jnp.concatenate([oh, x[:, 1:]], axis=1)


def _prep_body(x_ref, wd_ref, bd_ref, pts_ref, lsh_ref):
    enc = _encode(x_ref[0])
    xd = _selu(_mm(enc, wd_ref[...]) + bd_ref[...])
    lsh_ref[0] = xd[:, :_NBIN]
    pts_ref[0] = xd[:, _NBIN:]


def _mlp1_body(x_ref, W1, b1, W2, b2, W3, b3,
               c1Wt, c1bt, c1Wh, c1th,
               fh_ref, gate_ref, base_ref):
    enc = _encode(x_ref[0])
    x = _selu(_mm(enc, W1[...]) + b1[...])
    x = _selu(_mm(x, W2[...]) + b2[...])
    x = _selu(_mm(x, W3[...]) + b3[...])
    gate = jax.nn.sigmoid(_mm(x, c1Wt[...]) + c1bt[...])
    fh_ref[0] = _mm(x, c1th[...])
    gate_ref[0] = gate
    base_ref[0] = (1.0 - gate) * _mm(x, c1Wh[...])


_CHUNK = 128
_NGRP = _CHUNK // 16


def _route_body(lsh_hbm, m_hbm, lsh_v, cnt_v, offs_v, m_v, csh, acf):
    sample = jax.lax.axis_index("c")
    s = jax.lax.axis_index("s")
    base = sample * _N + s * _CHUNK
    pltpu.sync_copy(lsh_hbm.at[pl.ds(base * _NBIN, _CHUNK * _NBIN)], lsh_v)
    zeros16f = jnp.zeros((16,), jnp.float32)
    for q in range(_NBIN // 16):
        cnt_v[q * 16:(q + 1) * 16] = zeros16f
    iota16 = jax.lax.broadcasted_iota(jnp.int32, (16,), 0)
    lane0 = iota16 == 0
    bins_all = []
    ranks_all = []
    for g in range(_NGRP):
        rowoff = (iota16 + g * 16) * _NBIN
        v0 = plsc.load_gather(lsh_v, [rowoff])

        def cbody(c, carry, rowoff=rowoff):
            cur_max, cur_bin = carry
            cvec = jnp.zeros((16,), jnp.int32) + c
            v = plsc.load_gather(lsh_v, [rowoff + cvec])
            upd = v > cur_max
            return (jnp.where(upd, v, cur_max), jnp.where(upd, cvec, cur_bin))

        _, bins = jax.lax.fori_loop(
            1, _NBIN, cbody, (v0, jnp.zeros((16,), jnp.int32)))

        def bbody(b, prev, bins=bins):
            bvec = jnp.zeros((16,), jnp.int32) + b
            mask = bins == bvec
            mi = mask.astype(jnp.int32)
            pc = plsc.cumsum(mi)
            cntb = plsc.load_gather(cnt_v, [bvec])
            prev = jnp.where(
                mask, cntb + (pc - mi).astype(jnp.float32), prev)
            tot = plsc.all_reduce_population_count(mask)
            plsc.store_scatter(
                cnt_v, [bvec], cntb + tot.astype(jnp.float32), mask=lane0)
            return prev

        rank = jax.lax.fori_loop(0, _NBIN, bbody, zeros16f)
        bins_all.append(bins)
        ranks_all.append(rank)
    pltpu.sync_copy(cnt_v, csh.at[pl.ds(s * _NBIN, _NBIN)])
    plsc.subcore_barrier()
    pltpu.sync_copy(csh, acf)
    for q in range(_NBIN // 16):
        acc = zeros16f
        for j in range(16):
            f = jnp.where(j < s, 1.0, 0.0)
            acc = acc + f * acf[j * _NBIN + q * 16:j * _NBIN + (q + 1) * 16]
        offs_v[q * 16:(q + 1) * 16] = acc
    for g in range(_NGRP):
        off = plsc.load_gather(offs_v, [bins_all[g]])
        grank = ranks_all[g] + off
        mval = jnp.where(
            grank < float(_MPB), bins_all[g].astype(jnp.float32), -1.0)
        m_v[g * 16:(g + 1) * 16] = mval
    pltpu.sync_copy(m_v, m_hbm.at[pl.ds(base, _CHUNK)])


def _make_route(nb):
    return pl.kernel(
        _route_body,
        out_type=jax.ShapeDtypeStruct((nb * _N,), jnp.float32),
        mesh=plsc.VectorSubcoreMesh(
            core_axis_name="c", subcore_axis_name="s", num_cores=nb),
        scratch_types=[
            pltpu.VMEM((_CHUNK * _NBIN,), jnp.float32),
            pltpu.VMEM((_NBIN,), jnp.float32),
            pltpu.VMEM((_NBIN,), jnp.float32),
            pltpu.VMEM((_CHUNK,), jnp.float32),
            pltpu.VMEM_SHARED((16 * _NBIN,), jnp.float32),
            pltpu.VMEM((16 * _NBIN,), jnp.float32),
        ],
        compiler_params=pltpu.CompilerParams(needs_layout_passes=False),
    )


def _adj_body(ptsr_ref, mr_ref, pts_ref, mf_ref, adj_ref, deg_ref):
    rows = ptsr_ref[0]
    ptsf = pts_ref[0]
    mr = mr_ref[0]
    mf = mf_ref[0]
    ior = jax.lax.broadcasted_iota(
        jnp.int32, (_TILE, _NBIN), 1).astype(jnp.float32)
    iof = jax.lax.broadcasted_iota(
        jnp.int32, (_N, _NBIN), 1).astype(jnp.float32)
    bvr = (mr == ior).astype(jnp.float32)
    bvf = (mf == iof).astype(jnp.float32)
    vm = _dotT(bvr, bvf) > 0.5
    sim = _dotT(rows, ptsf)
    sim = jnp.where(vm, sim, -1e9)
    m = jnp.max(sim, axis=1, keepdims=True)
    e = jnp.exp(sim - m)
    d = e / jnp.sum(e, axis=1, keepdims=True)
    d = jnp.where(vm, d, 0.0)
    a = jnp.where(d > _CUT, jnp.exp(-d), 0.0)
    adj_ref[0] = a.astype(jnp.bfloat16)
    deg_ref[0] = jnp.sum(a, axis=1, keepdims=True)


def _gh_body(adj_ref, fh_ref, gate_ref, base_ref, degf_ref, degr_ref,
             out_ref):
    nf = jax.lax.rsqrt(degf_ref[0] + 1e-6)
    nr = jax.lax.rsqrt(degr_ref[0] + 1e-6)
    fhn = (fh_ref[0] * nf).astype(jnp.bfloat16)
    agg = _mm(adj_ref[0], fhn)
    out_ref[0] = gate_ref[0] * nr * agg + base_ref[0]


def _head_body(xg_ref,
               id1W, id1b, id2W, id2b, id3W, id3b,
               idW, idb, chW, chb,
               c2Wt, c2bt, c2Wh, c2th,
               lc_ref, fh_ref, gate_ref, base_ref):
    xg = xg_ref[0]
    a = _selu(_mm(xg, id1W[...]) + id1b[...])
    a = _selu(_mm(a, id2W[...]) + id2b[...])
    a = _selu(_mm(a, id3W[...]) + id3b[...])
    logits = _mm(a, idW[...]) + idb[...]
    charge = _mm(a, chW[...]) + chb[...]
    lm = jnp.max(logits, axis=1, keepdims=True)
    le = jnp.exp(logits - lm)
    sm = le / jnp.sum(le, axis=1, keepdims=True)
    x2 = jnp.concatenate([xg, sm], axis=1)
    gate = jax.nn.sigmoid(_mm(x2, c2Wt[...]) + c2bt[...])
    lc_ref[0] = jnp.concatenate([logits, charge], axis=1)
    fh_ref[0] = _mm(x2, c2th[...])
    gate_ref[0] = gate
    base_ref[0] = (1.0 - gate) * _mm(x2, c2Wh[...])


def _tail_body(x_ref, x2g_ref, lc_ref,
               m1W, m1b, m2W, m2b, m3W, m3b,
               mo1W, mo1b, mo2W, mo2b, mo3W, mo3b, moW, mob,
               out_ref):
    enc = _encode(x_ref[0])
    xm = _selu(_mm(enc, m1W[...]) + m1b[...])
    xm = _selu(_mm(xm, m2W[...]) + m2b[...])
    xm = _selu(_mm(xm, m3W[...]) + m3b[...])
    d = jnp.concatenate([x2g_ref[0], xm], axis=1)
    d = _selu(_mm(d, mo1W[...]) + mo1b[...])
    d = _selu(_mm(d, mo2W[...]) + mo2b[...])
    d = _selu(_mm(d, mo3W[...]) + mo3b[...])
    mom = _mm(d, moW[...]) + mob[...]
    out_ref[0] = jnp.concatenate([lc_ref[0], mom], axis=1)


def _wspec(arr):
    return pl.BlockSpec(arr.shape, lambda *_: (0,) * arr.ndim)


def _full(d):
    return pl.BlockSpec((1, _N, d), lambda b: (b, 0, 0))


def kernel(X, params):
    p = params
    B = X.shape[0]
    f32 = jnp.float32

    def rb(v):
        return v.reshape(1, -1)

    def shp(d):
        return jax.ShapeDtypeStruct((B, _N, d), f32)

    pts, lsh = pl.pallas_call(
        _prep_body,
        grid=(B,),
        in_specs=[_full(15), _wspec(p['Wd']),
                  pl.BlockSpec((1, 256), lambda b: (0, 0))],
        out_specs=[_full(192), _full(_NBIN)],
        out_shape=[shp(192), shp(_NBIN)],
    )(X, p['Wd'], rb(p['bd']))

    m = _make_route(B)(lsh.reshape(B * _N * _NBIN))
    m3 = m.reshape(B, _N, 1)

    w1 = [p['W1'], rb(p['b1']), p['W2'], rb(p['b2']), p['W3'], rb(p['b3']),
          p['c1_Wt'], rb(p['c1_bt']), p['c1_Wh'], p['c1_theta']]
    fh1, gate1, base1 = pl.pallas_call(
        _mlp1_body,
        grid=(B,),
        in_specs=[_full(15)] + [_wspec(w) for w in w1],
        out_specs=[_full(_D1), _full(_D1), _full(_D1)],
        out_shape=[shp(_D1), shp(_D1), shp(_D1)],
    )(X, *w1)

    adj, deg = pl.pallas_call(
        _adj_body,
        grid=(B, _NT),
        in_specs=[
            pl.BlockSpec((1, _TILE, 192), lambda b, t: (b, t, 0)),
            pl.BlockSpec((1, _TILE, 1), lambda b, t: (b, t, 0)),
            pl.BlockSpec((1, _N, 192), lambda b, t: (b, 0, 0)),
            pl.BlockSpec((1, _N, 1), lambda b, t: (b, 0, 0)),
        ],
        out_specs=[
            pl.BlockSpec((1, _TILE, _N), lambda b, t: (b, t, 0)),
            pl.BlockSpec((1, _TILE, 1), lambda b, t: (b, t, 0)),
        ],
        out_shape=[jax.ShapeDtypeStruct((B, _N, _N), jnp.bfloat16), shp(1)],
    )(pts, m3, pts, m3)

    def gh(adj, fh, gate, base, deg, d):
        return pl.pallas_call(
            _gh_body,
            grid=(B, _NT),
            in_specs=[
                pl.BlockSpec((1, _TILE, _N), lambda b, t: (b, t, 0)),
                pl.BlockSpec((1, _N, d), lambda b, t: (b, 0, 0)),
                pl.BlockSpec((1, _TILE, d), lambda b, t: (b, t, 0)),
                pl.BlockSpec((1, _TILE, d), lambda b, t: (b, t, 0)),
                pl.BlockSpec((1, _N, 1), lambda b, t: (b, 0, 0)),
                pl.BlockSpec((1, _TILE, 1), lambda b, t: (b, t, 0)),
            ],
            out_specs=pl.BlockSpec((1, _TILE, d), lambda b, t: (b, t, 0)),
            out_shape=shp(d),
        )(adj, fh, gate, base, deg, deg)

    xg = gh(adj, fh1, gate1, base1, deg, _D1)

    w2 = [p['id1_W'], rb(p['id1_b']), p['id2_W'], rb(p['id2_b']),
          p['id3_W'], rb(p['id3_b']), p['idW'], rb(p['idb']),
          p['chW'], rb(p['chb']),
          p['c2_Wt'], rb(p['c2_bt']), p['c2_Wh'], p['c2_theta']]
    lc, fh2, gate2, base2 = pl.pallas_call(
        _head_body,
        grid=(B,),
        in_specs=[_full(_D1)] + [_wspec(w) for w in w2],
        out_specs=[_full(9), _full(_D2), _full(_D2), _full(_D2)],
        out_shape=[shp(9), shp(_D2), shp(_D2), shp(_D2)],
    )(xg, *w2)

    x2g = gh(adj, fh2, gate2, base2, deg, _D2)

    w3 = [p['m1_W'], rb(p['m1_b']), p['m2_W'], rb(p['m2_b']),
          p['m3_W'], rb(p['m3_b']),
          p['mo1_W'], rb(p['mo1_b']), p['mo2_W'], rb(p['mo2_b']),
          p['mo3_W'], rb(p['mo3_b']), p['moW'], rb(p['mob'])]
    out = pl.pallas_call(
        _tail_body,
        grid=(B,),
        in_specs=[_full(15), _full(_D2), _full(9)] + [_wspec(w) for w in w3],
        out_specs=_full(12),
        out_shape=shp(12),
    )(X, x2g, lc, *w3)
    return out


# speedup vs baseline: 1.1395x; 1.0798x over previous
import functools

import jax
import jax.numpy as jnp
from jax.experimental import pallas as pl
from jax.experimental.pallas import tpu as pltpu
from jax.experimental.pallas import tpu_sc as plsc

_N = 2048
_NBIN = 64
_MPB = 256
_CUT = 0.2
_TILE = 256
_NT = _N // _TILE
_NCLS = 12
_D1 = 512
_D2 = 520

_SELU_SCALE = 1.0507009873554805
_SELU_ALPHA = 1.6732632423543772


def _selu(x):
    return _SELU_SCALE * jnp.where(x > 0, x, _SELU_ALPHA * (jnp.exp(x) - 1.0))


def _mm(a, b):
    return jnp.dot(a, b, preferred_element_type=jnp.float32)


def _dotT(a, b):
    return jax.lax.dot_general(
        a, b, (((1,), (1,)), ((), ())), preferred_element_type=jnp.float32)


def _encode(x):
    ids = x[:, 0:1].astype(jnp.int32)
    ioh = jax.lax.broadcasted_iota(jnp.int32, (_N, _NCLS), 1)
    oh = (ids == ioh).astype(jnp.float32)
    return jnp.concatenate([oh, x[:, 1:]], axis=1)


def _prep_body(x_ref, wd_ref, bd_ref, pts_ref, lsh_ref):
    enc = _encode(x_ref[0])
    xd = _selu(_mm(enc, wd_ref[...]) + bd_ref[...])
    lsh_ref[0] = xd[:, :_NBIN]
    pts_ref[0] = xd[:, _NBIN:]


def _mlp1_body(x_ref, W1, b1, W2, b2, W3, b3,
               c1Wt, c1bt, c1Wh, c1th,
               fh_ref, gate_ref, base_ref):
    enc = _encode(x_ref[0])
    x = _selu(_mm(enc, W1[...]) + b1[...])
    x = _selu(_mm(x, W2[...]) + b2[...])
    x = _selu(_mm(x, W3[...]) + b3[...])
    gate = jax.nn.sigmoid(_mm(x, c1Wt[...]) + c1bt[...])
    fh_ref[0] = _mm(x, c1th[...]).astype(jnp.bfloat16)
    gate_ref[0] = gate.astype(jnp.bfloat16)
    base_ref[0] = ((1.0 - gate) * _mm(x, c1Wh[...])).astype(jnp.bfloat16)


_CHUNK = 128
_NGRP = _CHUNK // 16


def _route_body(lsh_hbm, m_hbm, lsh_v, cnt_v, offs_v, m_v, csh, acf):
    sample = jax.lax.axis_index("c")
    s = jax.lax.axis_index("s")
    base = sample * _N + s * _CHUNK
    pltpu.sync_copy(lsh_hbm.at[pl.ds(base * _NBIN, _CHUNK * _NBIN)], lsh_v)
    zeros16f = jnp.zeros((16,), jnp.float32)
    for q in range(_NBIN // 16):
        cnt_v[q * 16:(q + 1) * 16] = zeros16f
    iota16 = jax.lax.broadcasted_iota(jnp.int32, (16,), 0)
    lane0 = iota16 == 0
    bins_all = []
    ranks_all = []
    for g in range(_NGRP):
        rowoff = (iota16 + g * 16) * _NBIN
        v0 = plsc.load_gather(lsh_v, [rowoff])

        def cbody(c, carry, rowoff=rowoff):
            cur_max, cur_bin = carry
            cvec = jnp.zeros((16,), jnp.int32) + c
            v = plsc.load_gather(lsh_v, [rowoff + cvec])
            upd = v > cur_max
            return (jnp.where(upd, v, cur_max), jnp.where(upd, cvec, cur_bin))

        _, bins = jax.lax.fori_loop(
            1, _NBIN, cbody, (v0, jnp.zeros((16,), jnp.int32)))

        def bbody(b, prev, bins=bins):
            bvec = jnp.zeros((16,), jnp.int32) + b
            mask = bins == bvec
            mi = mask.astype(jnp.int32)
            pc = plsc.cumsum(mi)
            cntb = plsc.load_gather(cnt_v, [bvec])
            prev = jnp.where(
                mask, cntb + (pc - mi).astype(jnp.float32), prev)
            tot = plsc.all_reduce_population_count(mask)
            plsc.store_scatter(
                cnt_v, [bvec], cntb + tot.astype(jnp.float32), mask=lane0)
            return prev

        rank = jax.lax.fori_loop(0, _NBIN, bbody, zeros16f)
        bins_all.append(bins)
        ranks_all.append(rank)
    pltpu.sync_copy(cnt_v, csh.at[pl.ds(s * _NBIN, _NBIN)])
    plsc.subcore_barrier()
    pltpu.sync_copy(csh, acf)
    for q in range(_NBIN // 16):
        acc = zeros16f
        for j in range(16):
            f = jnp.where(j < s, 1.0, 0.0)
            acc = acc + f * acf[j * _NBIN + q * 16:j * _NBIN + (q + 1) * 16]
        offs_v[q * 16:(q + 1) * 16] = acc
    for g in range(_NGRP):
        off = plsc.load_gather(offs_v, [bins_all[g]])
        grank = ranks_all[g] + off
        mval = jnp.where(
            grank < float(_MPB), bins_all[g].astype(jnp.float32), -1.0)
        m_v[g * 16:(g + 1) * 16] = mval
    pltpu.sync_copy(m_v, m_hbm.at[pl.ds(base, _CHUNK)])


def _make_route(nb):
    return pl.kernel(
        _route_body,
        out_type=jax.ShapeDtypeStruct((nb * _N,), jnp.float32),
        mesh=plsc.VectorSubcoreMesh(
            core_axis_name="c", subcore_axis_name="s", num_cores=nb),
        scratch_types=[
            pltpu.VMEM((_CHUNK * _NBIN,), jnp.float32),
            pltpu.VMEM((_NBIN,), jnp.float32),
            pltpu.VMEM((_NBIN,), jnp.float32),
            pltpu.VMEM((_CHUNK,), jnp.float32),
            pltpu.VMEM_SHARED((16 * _NBIN,), jnp.float32),
            pltpu.VMEM((16 * _NBIN,), jnp.float32),
        ],
        compiler_params=pltpu.CompilerParams(needs_layout_passes=False),
    )


def _adj_body(ptsr_ref, mr_ref, pts_ref, mf_ref, adj_ref, deg_ref):
    rows = ptsr_ref[0]
    ptsf = pts_ref[0]
    mr = mr_ref[0]
    mf = mf_ref[0]
    ior = jax.lax.broadcasted_iota(
        jnp.int32, (_TILE, _NBIN), 1).astype(jnp.float32)
    iof = jax.lax.broadcasted_iota(
        jnp.int32, (_N, _NBIN), 1).astype(jnp.float32)
    bvr = (mr == ior).astype(jnp.float32)
    bvf = (mf == iof).astype(jnp.float32)
    vm = _dotT(bvr, bvf) > 0.5
    sim = _dotT(rows, ptsf)
    sim = jnp.where(vm, sim, -1e9)
    m = jnp.max(sim, axis=1, keepdims=True)
    e = jnp.exp(sim - m)
    d = e / jnp.sum(e, axis=1, keepdims=True)
    d = jnp.where(vm, d, 0.0)
    a = jnp.where(d > _CUT, jnp.exp(-d), 0.0)
    adj_ref[0] = a.astype(jnp.bfloat16)
    deg_ref[0] = jnp.sum(a, axis=1, keepdims=True)


def _gh_body(adj_ref, fh_ref, gate_ref, base_ref, degf_ref, degr_ref,
             out_ref):
    nf = jax.lax.rsqrt(degf_ref[0] + 1e-6)
    nr = jax.lax.rsqrt(degr_ref[0] + 1e-6)
    fhn = (fh_ref[0].astype(jnp.float32) * nf).astype(jnp.bfloat16)
    agg = _mm(adj_ref[0], fhn)
    gate = gate_ref[0].astype(jnp.float32)
    base = base_ref[0].astype(jnp.float32)
    out_ref[0] = (gate * nr * agg + base).astype(jnp.bfloat16)


def _head_body(xg_ref,
               id1W, id1b, id2W, id2b, id3W, id3b,
               idW, idb, chW, chb,
               c2Wt, c2bt, c2Wh, c2th,
               lc_ref, fh_ref, gate_ref, base_ref):
    xg = xg_ref[0].astype(jnp.float32)
    a = _selu(_mm(xg, id1W[...]) + id1b[...])
    a = _selu(_mm(a, id2W[...]) + id2b[...])
    a = _selu(_mm(a, id3W[...]) + id3b[...])
    logits = _mm(a, idW[...]) + idb[...]
    charge = _mm(a, chW[...]) + chb[...]
    lm = jnp.max(logits, axis=1, keepdims=True)
    le = jnp.exp(logits - lm)
    sm = le / jnp.sum(le, axis=1, keepdims=True)
    x2 = jnp.concatenate([xg, sm], axis=1)
    gate = jax.nn.sigmoid(_mm(x2, c2Wt[...]) + c2bt[...])
    lc_ref[0] = jnp.concatenate([logits, charge], axis=1)
    fh_ref[0] = _mm(x2, c2th[...]).astype(jnp.bfloat16)
    gate_ref[0] = gate.astype(jnp.bfloat16)
    base_ref[0] = ((1.0 - gate) * _mm(x2, c2Wh[...])).astype(jnp.bfloat16)


def _tail_body(x_ref, x2g_ref, lc_ref,
               m1W, m1b, m2W, m2b, m3W, m3b,
               mo1W, mo1b, mo2W, mo2b, mo3W, mo3b, moW, mob,
               out_ref):
    enc = _encode(x_ref[0])
    xm = _selu(_mm(enc, m1W[...]) + m1b[...])
    xm = _selu(_mm(xm, m2W[...]) + m2b[...])
    xm = _selu(_mm(xm, m3W[...]) + m3b[...])
    d = jnp.concatenate([x2g_ref[0].astype(jnp.float32), xm], axis=1)
    d = _selu(_mm(d, mo1W[...]) + mo1b[...])
    d = _selu(_mm(d, mo2W[...]) + mo2b[...])
    d = _selu(_mm(d, mo3W[...]) + mo3b[...])
    mom = _mm(d, moW[...]) + mob[...]
    out_ref[0] = jnp.concatenate([lc_ref[0], mom], axis=1)


def _wspec(arr):
    return pl.BlockSpec(arr.shape, lambda *_: (0,) * arr.ndim)


def _full(d):
    return pl.BlockSpec((1, _N, d), lambda b: (b, 0, 0))


def kernel(X, params):
    p = params
    B = X.shape[0]
    f32 = jnp.float32

    def rb(v):
        return v.reshape(1, -1)

    def shp(d):
        return jax.ShapeDtypeStruct((B, _N, d), f32)

    def shpb(d):
        return jax.ShapeDtypeStruct((B, _N, d), jnp.bfloat16)

    pts, lsh = pl.pallas_call(
        _prep_body,
        grid=(B,),
        in_specs=[_full(15), _wspec(p['Wd']),
                  pl.BlockSpec((1, 256), lambda b: (0, 0))],
        out_specs=[_full(192), _full(_NBIN)],
        out_shape=[shp(192), shp(_NBIN)],
    )(X, p['Wd'], rb(p['bd']))

    m = _make_route(B)(lsh.reshape(B * _N * _NBIN))
    m3 = m.reshape(B, _N, 1)

    w1 = [p['W1'], rb(p['b1']), p['W2'], rb(p['b2']), p['W3'], rb(p['b3']),
          p['c1_Wt'], rb(p['c1_bt']), p['c1_Wh'], p['c1_theta']]
    fh1, gate1, base1 = pl.pallas_call(
        _mlp1_body,
        grid=(B,),
        in_specs=[_full(15)] + [_wspec(w) for w in w1],
        out_specs=[_full(_D1), _full(_D1), _full(_D1)],
        out_shape=[shpb(_D1), shpb(_D1), shpb(_D1)],
    )(X, *w1)

    adj, deg = pl.pallas_call(
        _adj_body,
        grid=(B, _NT),
        in_specs=[
            pl.BlockSpec((1, _TILE, 192), lambda b, t: (b, t, 0)),
            pl.BlockSpec((1, _TILE, 1), lambda b, t: (b, t, 0)),
            pl.BlockSpec((1, _N, 192), lambda b, t: (b, 0, 0)),
            pl.BlockSpec((1, _N, 1), lambda b, t: (b, 0, 0)),
        ],
        out_specs=[
            pl.BlockSpec((1, _TILE, _N), lambda b, t: (b, t, 0)),
            pl.BlockSpec((1, _TILE, 1), lambda b, t: (b, t, 0)),
        ],
        out_shape=[jax.ShapeDtypeStruct((B, _N, _N), jnp.bfloat16), shp(1)],
    )(pts, m3, pts, m3)

    def gh(adj, fh, gate, base, deg, d):
        return pl.pallas_call(
            _gh_body,
            grid=(B, _NT),
            in_specs=[
                pl.BlockSpec((1, _TILE, _N), lambda b, t: (b, t, 0)),
                pl.BlockSpec((1, _N, d), lambda b, t: (b, 0, 0)),
                pl.BlockSpec((1, _TILE, d), lambda b, t: (b, t, 0)),
                pl.BlockSpec((1, _TILE, d), lambda b, t: (b, t, 0)),
                pl.BlockSpec((1, _N, 1), lambda b, t: (b, 0, 0)),
                pl.BlockSpec((1, _TILE, 1), lambda b, t: (b, t, 0)),
            ],
            out_specs=pl.BlockSpec((1, _TILE, d), lambda b, t: (b, t, 0)),
            out_shape=shpb(d),
        )(adj, fh, gate, base, deg, deg)

    xg = gh(adj, fh1, gate1, base1, deg, _D1)

    w2 = [p['id1_W'], rb(p['id1_b']), p['id2_W'], rb(p['id2_b']),
          p['id3_W'], rb(p['id3_b']), p['idW'], rb(p['idb']),
          p['chW'], rb(p['chb']),
          p['c2_Wt'], rb(p['c2_bt']), p['c2_Wh'], p['c2_theta']]
    lc, fh2, gate2, base2 = pl.pallas_call(
        _head_body,
        grid=(B,),
        in_specs=[_full(_D1)] + [_wspec(w) for w in w2],
        out_specs=[_full(9), _full(_D2), _full(_D2), _full(_D2)],
        out_shape=[shp(9), shpb(_D2), shpb(_D2), shpb(_D2)],
    )(xg, *w2)

    x2g = gh(adj, fh2, gate2, base2, deg, _D2)

    w3 = [p['m1_W'], rb(p['m1_b']), p['m2_W'], rb(p['m2_b']),
          p['m3_W'], rb(p['m3_b']),
          p['mo1_W'], rb(p['mo1_b']), p['mo2_W'], rb(p['mo2_b']),
          p['mo3_W'], rb(p['mo3_b']), p['moW'], rb(p['mob'])]
    out = pl.pallas_call(
        _tail_body,
        grid=(B,),
        in_specs=[_full(15), _full(_D2), _full(9)] + [_wspec(w) for w in w3],
        out_specs=_full(12),
        out_shape=shp(12),
    )(X, x2g, lc, *w3)
    return out
